# baseline (speedup 1.0000x reference)
import functools

import jax
import jax.numpy as jnp
from jax import lax
from jax.experimental import pallas as pl
from jax.experimental.pallas import tpu as pltpu
from jax.experimental.pallas import tpu_sc as plsc

BATCH = 4096
SEQ = 200
EMBED = 64
NUM_CORES = 2
NUM_SUBCORES = 16
NUM_WORKERS = NUM_CORES * NUM_SUBCORES
SEQS_PER_WORKER = BATCH // NUM_WORKERS
LANES = 16


def _body(idx_hbm, tok_hbm, pos_hbm, out_hbm, idx_v, rows_v, pos_v, sem):
    wid = lax.axis_index("s") * NUM_CORES + lax.axis_index("c")

    pltpu.sync_copy(pos_hbm, pos_v)

    def seq_body(s, _):
        base = pl.multiple_of((wid * SEQS_PER_WORKER + s) * SEQ, SEQ)
        pltpu.sync_copy(idx_hbm.at[pl.ds(base, SEQ)], idx_v)
        cp0 = pltpu.async_copy(
            tok_hbm.at[idx_v.at[pl.ds(0, 128)]], rows_v.at[pl.ds(0, 128)], sem
        )
        cp1 = pltpu.async_copy(
            tok_hbm.at[idx_v.at[pl.ds(128, SEQ - 128)]],
            rows_v.at[pl.ds(128, SEQ - 128)],
            sem,
        )
        cp0.wait()
        cp1.wait()

        def add_row(r, _):
            for c in range(EMBED // LANES):
                sl = (r, pl.ds(c * LANES, LANES))
                rows_v[sl] = rows_v[sl] + pos_v[sl]
            return ()

        lax.fori_loop(0, SEQ, add_row, ())
        pltpu.sync_copy(rows_v, out_hbm.at[pl.ds(base, SEQ)])
        return ()

    lax.fori_loop(0, SEQS_PER_WORKER, seq_body, ())


@jax.jit
def kernel(inputs, token_table, position_table):
    idx_flat = inputs.reshape(-1).astype(jnp.int32)
    mesh = plsc.VectorSubcoreMesh(
        core_axis_name="c", subcore_axis_name="s", num_cores=NUM_CORES,
        num_subcores=NUM_SUBCORES,
    )
    out_flat = pl.kernel(
        _body,
        out_type=jax.ShapeDtypeStruct((BATCH * SEQ, EMBED), jnp.float32),
        mesh=mesh,
        scratch_types=[
            pltpu.VMEM((SEQ,), jnp.int32),
            pltpu.VMEM((SEQ, EMBED), jnp.float32),
            pltpu.VMEM((SEQ, EMBED), jnp.float32),
            pltpu.SemaphoreType.DMA,
        ],
        compiler_params=pltpu.CompilerParams(use_tc_tiling_on_sc=False),
    )(idx_flat, token_table, position_table)
    return out_flat.reshape(BATCH, SEQ, EMBED)
